# baseline (speedup 1.0000x reference)
import functools

import jax
import jax.numpy as jnp
from jax.experimental import pallas as pl
from jax.experimental.pallas import tpu as pltpu
from jax.experimental.shard_map import shard_map
from jax.sharding import Mesh, PartitionSpec as P

N = 4096
BLK = 512
ALPHA = 0.2
LOG2E = 1.4426950408889634


def _prep_kernel(x_ref, w_ref, a_ref, haug_ref, c_ref, d_ref, f2_ref,
                 g_ref, cs_ref, *, f):
    h = jnp.dot(x_ref[...], w_ref[...], preferred_element_type=jnp.float32)
    haug_ref[:, :f] = h
    haug_ref[:, f:] = jnp.ones((N, 1), jnp.float32)
    f1 = jnp.dot(h, a_ref[:f, :], preferred_element_type=jnp.float32)
    f2 = jax.lax.dot_general(
        a_ref[f:, :], h, (((0,), (1,)), ((), ())),
        preferred_element_type=jnp.float32)
    lg = jnp.float32(LOG2E)
    f2_ref[...] = f2 * lg
    g_ref[...] = f2 * jnp.float32(ALPHA * LOG2E)
    m2 = jnp.max(f2)
    t = f1 + m2
    m = jnp.maximum(t, t * jnp.float32(ALPHA))
    c_ref[...] = (f1 - m) * lg
    d_ref[...] = f1 * jnp.float32(ALPHA * LOG2E) - m * lg
    cs_ref[...] = jnp.sum(h, axis=0, keepdims=True)


def _prep(x, w, a):
    f = w.shape[1]
    din = x.shape[1]
    return pl.pallas_call(
        functools.partial(_prep_kernel, f=f),
        grid=(1,),
        in_specs=[
            pl.BlockSpec((N, din), lambda i: (0, 0)),
            pl.BlockSpec((din, f), lambda i: (0, 0)),
            pl.BlockSpec((2 * f, 1), lambda i: (0, 0)),
        ],
        out_specs=[
            pl.BlockSpec((N, f + 1), lambda i: (0, 0)),
            pl.BlockSpec((N, 1), lambda i: (0, 0)),
            pl.BlockSpec((N, 1), lambda i: (0, 0)),
            pl.BlockSpec((1, N), lambda i: (0, 0)),
            pl.BlockSpec((1, N), lambda i: (0, 0)),
            pl.BlockSpec((1, f), lambda i: (0, 0)),
        ],
        out_shape=[
            jax.ShapeDtypeStruct((N, f + 1), jnp.float32),
            jax.ShapeDtypeStruct((N, 1), jnp.float32),
            jax.ShapeDtypeStruct((N, 1), jnp.float32),
            jax.ShapeDtypeStruct((1, N), jnp.float32),
            jax.ShapeDtypeStruct((1, N), jnp.float32),
            jax.ShapeDtypeStruct((1, f), jnp.float32),
        ],
    )(x, w, a)


def _softmax_block(adj_ref, haug_ref, c_ref, d_ref, f2_ref, g_ref, cs_ref, f):
    q = jnp.maximum(c_ref[...] + f2_ref[...], d_ref[...] + g_ref[...])
    pm = jnp.exp2(q) * adj_ref[...]
    hpz = jnp.dot(pm, haug_ref[...], preferred_element_type=jnp.float32)
    s = hpz[:, f:]
    z = jnp.where(s == 0, jnp.float32(1.0), jnp.float32(0.0))
    r = jnp.float32(1.0) / (s + z * jnp.float32(N))
    hp = (hpz[:, :f] + z * cs_ref[...]) * r
    return pm, z, r, hp


def _layer1_kernel(adj_ref, haug_ref, c_ref, d_ref, f2_ref, g_ref, cs_ref,
                   out_ref, *, f):
    _, _, _, hp = _softmax_block(adj_ref, haug_ref, c_ref, d_ref, f2_ref,
                                 g_ref, cs_ref, f)
    out_ref[...] = jnp.maximum(hp, 0.0)


def _layer2_kernel(adj_ref, haug_ref, c_ref, d_ref, f2_ref, g_ref, cs_ref,
                   e2c_ref, b_ref, attn_ref, logits_ref, *, f):
    pm, z, r, hp = _softmax_block(adj_ref, haug_ref, c_ref, d_ref, f2_ref,
                                  g_ref, cs_ref, f)
    attn_ref[...] = (pm + z) * r
    logits_ref[...] = (
        jnp.dot(hp, e2c_ref[...], preferred_element_type=jnp.float32)
        + b_ref[...])


def _common_specs(f):
    return [
        pl.BlockSpec((BLK, N), lambda i: (i, 0)),
        pl.BlockSpec((N, f + 1), lambda i: (0, 0)),
        pl.BlockSpec((BLK, 1), lambda i: (i, 0)),
        pl.BlockSpec((BLK, 1), lambda i: (i, 0)),
        pl.BlockSpec((1, N), lambda i: (0, 0)),
        pl.BlockSpec((1, N), lambda i: (0, 0)),
        pl.BlockSpec((1, f), lambda i: (0, 0)),
    ]


def _layer1(adj, haug, c, d, f2, g, cs):
    rows = adj.shape[0]
    f = haug.shape[1] - 1
    return pl.pallas_call(
        functools.partial(_layer1_kernel, f=f),
        grid=(rows // BLK,),
        in_specs=_common_specs(f),
        out_specs=pl.BlockSpec((BLK, f), lambda i: (i, 0)),
        out_shape=jax.ShapeDtypeStruct((rows, f), jnp.float32),
        compiler_params=pltpu.CompilerParams(
            dimension_semantics=("parallel",)),
    )(adj, haug, c, d, f2, g, cs)


def _layer2(adj, haug, c, d, f2, g, cs, e2c, b):
    rows = adj.shape[0]
    f = haug.shape[1] - 1
    out = e2c.shape[1]
    return pl.pallas_call(
        functools.partial(_layer2_kernel, f=f),
        grid=(rows // BLK,),
        in_specs=_common_specs(f) + [
            pl.BlockSpec((f, out), lambda i: (0, 0)),
            pl.BlockSpec((1, out), lambda i: (0, 0)),
        ],
        out_specs=[
            pl.BlockSpec((BLK, N), lambda i: (i, 0)),
            pl.BlockSpec((BLK, out), lambda i: (i, 0)),
        ],
        out_shape=[
            jax.ShapeDtypeStruct((rows, N), jnp.float32),
            jax.ShapeDtypeStruct((rows, out), jnp.float32),
        ],
        compiler_params=pltpu.CompilerParams(
            dimension_semantics=("parallel",)),
    )(adj, haug, c, d, f2, g, cs, e2c, b)


def _pipeline_shard(ft, adj, W1, a1, W2, a2, e2c, bias):
    haug, c, d, f2, g, cs = _prep(ft, W1, a1)
    i = jax.lax.axis_index("x")
    rows = adj.shape[0]
    cl = jax.lax.dynamic_slice(c, (i * rows, 0), (rows, 1))
    dl = jax.lax.dynamic_slice(d, (i * rows, 0), (rows, 1))
    embd_loc = _layer1(adj, haug, cl, dl, f2, g, cs)
    embd = jax.lax.all_gather(embd_loc, "x", axis=0, tiled=True)
    haug2, c2, d2, f22, g2, cs2 = _prep(embd, W2, a2)
    c2l = jax.lax.dynamic_slice(c2, (i * rows, 0), (rows, 1))
    d2l = jax.lax.dynamic_slice(d2, (i * rows, 0), (rows, 1))
    attention, logits = _layer2(adj, haug2, c2l, d2l, f22, g2, cs2,
                                e2c, bias)
    return (logits, embd, attention)


_CACHED = {}


def _build():
    devs = jax.devices()
    ndev = 2 if len(devs) >= 2 else 1
    if ndev in _CACHED:
        return _CACHED[ndev]
    mesh = Mesh(devs[:ndev], ("x",))
    fn = jax.jit(shard_map(
        _pipeline_shard, mesh=mesh,
        in_specs=(P(), P("x", None), P(), P(), P(), P(), P(), P()),
        out_specs=(P("x", None), P(), P("x", None)),
        check_rep=False,
    ))
    _CACHED[ndev] = fn
    return fn


def kernel(ft, adj, W1, a1, W2, a2, embd2class, bias):
    return _build()(ft, adj, W1, a1, W2, a2, embd2class, bias)


# speedup vs baseline: 5.4413x; 5.4413x over previous
import functools

import jax
import jax.numpy as jnp
from jax.experimental import pallas as pl
from jax.experimental.pallas import tpu as pltpu

N = 4096
BLK1 = 1024
BLK2 = 512
ALPHA = 0.2
LOG2E = 1.4426950408889634


def _prep_kernel(x_ref, w_ref, a_ref, haug_ref, c_ref, d_ref, f2_ref,
                 g_ref, cs_ref, *, f):
    h = jnp.dot(x_ref[...], w_ref[...], preferred_element_type=jnp.float32)
    haug_ref[:, :f] = h
    haug_ref[:, f:] = jnp.ones((N, 1), jnp.float32)
    f1 = jnp.dot(h, a_ref[:f, :], preferred_element_type=jnp.float32)
    f2 = jax.lax.dot_general(
        a_ref[f:, :], h, (((0,), (1,)), ((), ())),
        preferred_element_type=jnp.float32)
    lg = jnp.float32(LOG2E)
    f2_ref[...] = f2 * lg
    g_ref[...] = f2 * jnp.float32(ALPHA * LOG2E)
    m2 = jnp.max(f2)
    t = f1 + m2
    m = jnp.maximum(t, t * jnp.float32(ALPHA))
    c_ref[...] = (f1 - m) * lg
    d_ref[...] = f1 * jnp.float32(ALPHA * LOG2E) - m * lg
    cs_ref[...] = jnp.sum(h, axis=0, keepdims=True)


def _prep(x, w, a):
    f = w.shape[1]
    din = x.shape[1]
    return pl.pallas_call(
        functools.partial(_prep_kernel, f=f),
        grid=(1,),
        in_specs=[
            pl.BlockSpec((N, din), lambda i: (0, 0)),
            pl.BlockSpec((din, f), lambda i: (0, 0)),
            pl.BlockSpec((2 * f, 1), lambda i: (0, 0)),
        ],
        out_specs=[
            pl.BlockSpec((N, f + 1), lambda i: (0, 0)),
            pl.BlockSpec((N, 1), lambda i: (0, 0)),
            pl.BlockSpec((N, 1), lambda i: (0, 0)),
            pl.BlockSpec((1, N), lambda i: (0, 0)),
            pl.BlockSpec((1, N), lambda i: (0, 0)),
            pl.BlockSpec((1, f), lambda i: (0, 0)),
        ],
        out_shape=[
            jax.ShapeDtypeStruct((N, f + 1), jnp.float32),
            jax.ShapeDtypeStruct((N, 1), jnp.float32),
            jax.ShapeDtypeStruct((N, 1), jnp.float32),
            jax.ShapeDtypeStruct((1, N), jnp.float32),
            jax.ShapeDtypeStruct((1, N), jnp.float32),
            jax.ShapeDtypeStruct((1, f), jnp.float32),
        ],
    )(x, w, a)


def _softmax_block(adj_ref, haug_ref, c_ref, d_ref, f2_ref, g_ref, cs_ref, f):
    q = jnp.maximum(c_ref[...] + f2_ref[...], d_ref[...] + g_ref[...])
    pm = jnp.exp2(q) * adj_ref[...]
    hpz = jnp.dot(pm, haug_ref[...], preferred_element_type=jnp.float32)
    s = hpz[:, f:]
    z = jnp.where(s == 0, jnp.float32(1.0), jnp.float32(0.0))
    r = jnp.float32(1.0) / (s + z * jnp.float32(N))
    hp = (hpz[:, :f] + z * cs_ref[...]) * r
    return pm, z, r, hp


def _layer1_kernel(adj_ref, haug_ref, c_ref, d_ref, f2_ref, g_ref, cs_ref,
                   out_ref, *, f):
    _, _, _, hp = _softmax_block(adj_ref, haug_ref, c_ref, d_ref, f2_ref,
                                 g_ref, cs_ref, f)
    out_ref[...] = jnp.maximum(hp, 0.0)


def _layer2_kernel(adj_ref, haug_ref, c_ref, d_ref, f2_ref, g_ref, cs_ref,
                   e2c_ref, b_ref, attn_ref, logits_ref, *, f):
    pm, z, r, hp = _softmax_block(adj_ref, haug_ref, c_ref, d_ref, f2_ref,
                                  g_ref, cs_ref, f)
    attn_ref[...] = (pm + z) * r
    logits_ref[...] = (
        jnp.dot(hp, e2c_ref[...], preferred_element_type=jnp.float32)
        + b_ref[...])


def _common_specs(f, blk):
    return [
        pl.BlockSpec((blk, N), lambda i: (i, 0)),
        pl.BlockSpec((N, f + 1), lambda i: (0, 0)),
        pl.BlockSpec((blk, 1), lambda i: (i, 0)),
        pl.BlockSpec((blk, 1), lambda i: (i, 0)),
        pl.BlockSpec((1, N), lambda i: (0, 0)),
        pl.BlockSpec((1, N), lambda i: (0, 0)),
        pl.BlockSpec((1, f), lambda i: (0, 0)),
    ]


def _layer1(adj, haug, c, d, f2, g, cs):
    f = haug.shape[1] - 1
    return pl.pallas_call(
        functools.partial(_layer1_kernel, f=f),
        grid=(N // BLK1,),
        in_specs=_common_specs(f, BLK1),
        out_specs=pl.BlockSpec((BLK1, f), lambda i: (i, 0)),
        out_shape=jax.ShapeDtypeStruct((N, f), jnp.float32),
        compiler_params=pltpu.CompilerParams(
            dimension_semantics=("parallel",)),
    )(adj, haug, c, d, f2, g, cs)


def _layer2(adj, haug, c, d, f2, g, cs, e2c, b):
    f = haug.shape[1] - 1
    out = e2c.shape[1]
    return pl.pallas_call(
        functools.partial(_layer2_kernel, f=f),
        grid=(N // BLK2,),
        in_specs=_common_specs(f, BLK2) + [
            pl.BlockSpec((f, out), lambda i: (0, 0)),
            pl.BlockSpec((1, out), lambda i: (0, 0)),
        ],
        out_specs=[
            pl.BlockSpec((BLK2, N), lambda i: (i, 0)),
            pl.BlockSpec((BLK2, out), lambda i: (i, 0)),
        ],
        out_shape=[
            jax.ShapeDtypeStruct((N, N), jnp.float32),
            jax.ShapeDtypeStruct((N, out), jnp.float32),
        ],
        compiler_params=pltpu.CompilerParams(
            dimension_semantics=("parallel",)),
    )(adj, haug, c, d, f2, g, cs, e2c, b)


@jax.jit
def kernel(ft, adj, W1, a1, W2, a2, embd2class, bias):
    haug, c, d, f2, g, cs = _prep(ft, W1, a1)
    embd = _layer1(adj, haug, c, d, f2, g, cs)
    haug2, c2, d2, f22, g2, cs2 = _prep(embd, W2, a2)
    attention, logits = _layer2(adj, haug2, c2, d2, f22, g2, cs2,
                                embd2class, bias)
    return (logits, embd, attention)


# speedup vs baseline: 6.0212x; 1.1066x over previous
import functools

import jax
import jax.numpy as jnp
from jax.experimental import pallas as pl
from jax.experimental.pallas import tpu as pltpu

N = 4096
BLK1 = 512
BLK2 = 512
ALPHA = 0.2
LOG2E = 1.4426950408889634


def _prep_kernel(x_ref, w_ref, a_ref, haug_ref, c_ref, d_ref, f2_ref,
                 g_ref, cs_ref, *, f):
    h = jnp.dot(x_ref[...], w_ref[...], preferred_element_type=jnp.float32)
    haug_ref[:, :f] = h
    haug_ref[:, f:] = jnp.ones((N, 1), jnp.float32)
    f1 = jnp.dot(h, a_ref[:f, :], preferred_element_type=jnp.float32)
    f2 = jax.lax.dot_general(
        a_ref[f:, :], h, (((0,), (1,)), ((), ())),
        preferred_element_type=jnp.float32)
    lg = jnp.float32(LOG2E)
    f2_ref[...] = f2 * lg
    g_ref[...] = f2 * jnp.float32(ALPHA * LOG2E)
    m2 = jnp.max(f2)
    t = f1 + m2
    m = jnp.maximum(t, t * jnp.float32(ALPHA))
    c_ref[...] = (f1 - m) * lg
    d_ref[...] = f1 * jnp.float32(ALPHA * LOG2E) - m * lg
    cs_ref[...] = jnp.sum(h, axis=0, keepdims=True)


def _prep(x, w, a):
    f = w.shape[1]
    din = x.shape[1]
    return pl.pallas_call(
        functools.partial(_prep_kernel, f=f),
        grid=(1,),
        in_specs=[
            pl.BlockSpec((N, din), lambda i: (0, 0)),
            pl.BlockSpec((din, f), lambda i: (0, 0)),
            pl.BlockSpec((2 * f, 1), lambda i: (0, 0)),
        ],
        out_specs=[
            pl.BlockSpec((N, f + 1), lambda i: (0, 0)),
            pl.BlockSpec((N, 1), lambda i: (0, 0)),
            pl.BlockSpec((N, 1), lambda i: (0, 0)),
            pl.BlockSpec((1, N), lambda i: (0, 0)),
            pl.BlockSpec((1, N), lambda i: (0, 0)),
            pl.BlockSpec((1, f), lambda i: (0, 0)),
        ],
        out_shape=[
            jax.ShapeDtypeStruct((N, f + 1), jnp.float32),
            jax.ShapeDtypeStruct((N, 1), jnp.float32),
            jax.ShapeDtypeStruct((N, 1), jnp.float32),
            jax.ShapeDtypeStruct((1, N), jnp.float32),
            jax.ShapeDtypeStruct((1, N), jnp.float32),
            jax.ShapeDtypeStruct((1, f), jnp.float32),
        ],
    )(x, w, a)


def _softmax_block(mask, haug_ref, c_ref, d_ref, f2_ref, g_ref, cs_ref, f):
    q = jnp.maximum(c_ref[...] + f2_ref[...], d_ref[...] + g_ref[...])
    pm = jnp.exp2(q) * mask
    hpz = jnp.dot(pm, haug_ref[...], preferred_element_type=jnp.float32)
    s = hpz[:, f:]
    z = jnp.where(s == 0, jnp.float32(1.0), jnp.float32(0.0))
    r = jnp.float32(1.0) / (s + z * jnp.float32(N))
    hp = (hpz[:, :f] + z * cs_ref[...]) * r
    return pm, z, r, hp


def _layer1_kernel(adj_ref, haug_ref, c_ref, d_ref, f2_ref, g_ref, cs_ref,
                   out_ref, m8_ref, *, f):
    _, _, _, hp = _softmax_block(adj_ref[...], haug_ref, c_ref, d_ref,
                                 f2_ref, g_ref, cs_ref, f)
    out_ref[...] = jnp.maximum(hp, 0.0)
    m8_ref[...] = adj_ref[...].astype(jnp.int8)


def _layer2_kernel(m8_ref, haug_ref, c_ref, d_ref, f2_ref, g_ref, cs_ref,
                   e2c_ref, b_ref, attn_ref, logits_ref, *, f):
    pm, z, r, hp = _softmax_block(m8_ref[...].astype(jnp.float32), haug_ref,
                                  c_ref, d_ref, f2_ref, g_ref, cs_ref, f)
    attn_ref[...] = (pm + z) * r
    logits_ref[...] = (
        jnp.dot(hp, e2c_ref[...], preferred_element_type=jnp.float32)
        + b_ref[...])


def _common_specs(f, blk):
    return [
        pl.BlockSpec((blk, N), lambda i: (i, 0)),
        pl.BlockSpec((N, f + 1), lambda i: (0, 0)),
        pl.BlockSpec((blk, 1), lambda i: (i, 0)),
        pl.BlockSpec((blk, 1), lambda i: (i, 0)),
        pl.BlockSpec((1, N), lambda i: (0, 0)),
        pl.BlockSpec((1, N), lambda i: (0, 0)),
        pl.BlockSpec((1, f), lambda i: (0, 0)),
    ]


def _layer1(adj, haug, c, d, f2, g, cs):
    f = haug.shape[1] - 1
    return pl.pallas_call(
        functools.partial(_layer1_kernel, f=f),
        grid=(N // BLK1,),
        in_specs=_common_specs(f, BLK1),
        out_specs=[
            pl.BlockSpec((BLK1, f), lambda i: (i, 0)),
            pl.BlockSpec((BLK1, N), lambda i: (i, 0)),
        ],
        out_shape=[
            jax.ShapeDtypeStruct((N, f), jnp.float32),
            jax.ShapeDtypeStruct((N, N), jnp.int8),
        ],
        compiler_params=pltpu.CompilerParams(
            dimension_semantics=("parallel",)),
    )(adj, haug, c, d, f2, g, cs)


def _layer2(m8, haug, c, d, f2, g, cs, e2c, b):
    f = haug.shape[1] - 1
    out = e2c.shape[1]
    return pl.pallas_call(
        functools.partial(_layer2_kernel, f=f),
        grid=(N // BLK2,),
        in_specs=_common_specs(f, BLK2) + [
            pl.BlockSpec((f, out), lambda i: (0, 0)),
            pl.BlockSpec((1, out), lambda i: (0, 0)),
        ],
        out_specs=[
            pl.BlockSpec((BLK2, N), lambda i: (i, 0)),
            pl.BlockSpec((BLK2, out), lambda i: (i, 0)),
        ],
        out_shape=[
            jax.ShapeDtypeStruct((N, N), jnp.float32),
            jax.ShapeDtypeStruct((N, out), jnp.float32),
        ],
        compiler_params=pltpu.CompilerParams(
            dimension_semantics=("parallel",)),
    )(m8, haug, c, d, f2, g, cs, e2c, b)


@jax.jit
def kernel(ft, adj, W1, a1, W2, a2, embd2class, bias):
    haug, c, d, f2, g, cs = _prep(ft, W1, a1)
    embd, m8 = _layer1(adj, haug, c, d, f2, g, cs)
    haug2, c2, d2, f22, g2, cs2 = _prep(embd, W2, a2)
    attention, logits = _layer2(m8, haug2, c2, d2, f22, g2, cs2,
                                embd2class, bias)
    return (logits, embd, attention)


# speedup vs baseline: 6.9809x; 1.1594x over previous
import functools

import jax
import jax.numpy as jnp
from jax.experimental import pallas as pl
from jax.experimental.pallas import tpu as pltpu

N = 4096
BLK = 512
ALPHA = 0.2
LOG2E = 1.4426950408889634


def _prep_to_scratch(x_ref, w_ref, a_ref, haug_s, c_s, d_s, f2_s, g_s,
                     cs_s, f):
    h = jnp.dot(x_ref[...], w_ref[...], preferred_element_type=jnp.float32)
    haug_s[:, :f] = h
    haug_s[:, f:] = jnp.ones((N, 1), jnp.float32)
    f1 = jnp.dot(h, a_ref[:f, :], preferred_element_type=jnp.float32)
    f2 = jax.lax.dot_general(
        a_ref[f:, :], h, (((0,), (1,)), ((), ())),
        preferred_element_type=jnp.float32)
    lg = jnp.float32(LOG2E)
    f2_s[...] = f2 * lg
    g_s[...] = f2 * jnp.float32(ALPHA * LOG2E)
    m2 = jnp.max(f2)
    t = f1 + m2
    m = jnp.maximum(t, t * jnp.float32(ALPHA))
    c_s[...] = (f1 - m) * lg
    d_s[...] = f1 * jnp.float32(ALPHA * LOG2E) - m * lg
    cs_s[...] = jnp.sum(h, axis=0, keepdims=True)


def _softmax_block(mask, i, haug_s, c_s, d_s, f2_s, g_s, cs_s, f):
    c = c_s[pl.ds(i * BLK, BLK), :]
    d = d_s[pl.ds(i * BLK, BLK), :]
    q = jnp.maximum(c + f2_s[...], d + g_s[...])
    pm = jnp.exp2(q) * mask
    hpz = jnp.dot(pm, haug_s[...], preferred_element_type=jnp.float32)
    s = hpz[:, f:]
    z = jnp.where(s == 0, jnp.float32(1.0), jnp.float32(0.0))
    r = jnp.float32(1.0) / (s + z * jnp.float32(N))
    hp = (hpz[:, :f] + z * cs_s[...]) * r
    return pm, z, r, hp


def _layer1_kernel(x_ref, w_ref, a_ref, adj_ref, out_ref, m8_ref,
                   haug_s, c_s, d_s, f2_s, g_s, cs_s, *, f):
    i = pl.program_id(0)

    @pl.when(i == 0)
    def _():
        _prep_to_scratch(x_ref, w_ref, a_ref, haug_s, c_s, d_s, f2_s,
                         g_s, cs_s, f)

    _, _, _, hp = _softmax_block(adj_ref[...], i, haug_s, c_s, d_s, f2_s,
                                 g_s, cs_s, f)
    out_ref[...] = jnp.maximum(hp, 0.0)
    m8_ref[...] = adj_ref[...].astype(jnp.int8)


def _layer2_kernel(x_ref, w_ref, a_ref, m8_ref, e2c_ref, b_ref,
                   attn_ref, logits_ref,
                   haug_s, c_s, d_s, f2_s, g_s, cs_s, *, f):
    i = pl.program_id(0)

    @pl.when(i == 0)
    def _():
        _prep_to_scratch(x_ref, w_ref, a_ref, haug_s, c_s, d_s, f2_s,
                         g_s, cs_s, f)

    pm, z, r, hp = _softmax_block(m8_ref[...].astype(jnp.float32), i,
                                  haug_s, c_s, d_s, f2_s, g_s, cs_s, f)
    attn_ref[...] = (pm + z) * r
    logits_ref[...] = (
        jnp.dot(hp, e2c_ref[...], preferred_element_type=jnp.float32)
        + b_ref[...])


def _scratch(f):
    return [
        pltpu.VMEM((N, f + 1), jnp.float32),
        pltpu.VMEM((N, 1), jnp.float32),
        pltpu.VMEM((N, 1), jnp.float32),
        pltpu.VMEM((1, N), jnp.float32),
        pltpu.VMEM((1, N), jnp.float32),
        pltpu.VMEM((1, f), jnp.float32),
    ]


def _layer1(ft, W, a, adj):
    f = W.shape[1]
    din = ft.shape[1]
    return pl.pallas_call(
        functools.partial(_layer1_kernel, f=f),
        grid=(N // BLK,),
        in_specs=[
            pl.BlockSpec((N, din), lambda i: (0, 0)),
            pl.BlockSpec((din, f), lambda i: (0, 0)),
            pl.BlockSpec((2 * f, 1), lambda i: (0, 0)),
            pl.BlockSpec((BLK, N), lambda i: (i, 0)),
        ],
        out_specs=[
            pl.BlockSpec((BLK, f), lambda i: (i, 0)),
            pl.BlockSpec((BLK, N), lambda i: (i, 0)),
        ],
        out_shape=[
            jax.ShapeDtypeStruct((N, f), jnp.float32),
            jax.ShapeDtypeStruct((N, N), jnp.int8),
        ],
        scratch_shapes=_scratch(f),
    )(ft, W, a, adj)


def _layer2(x, W, a, m8, e2c, b):
    f = W.shape[1]
    din = x.shape[1]
    out = e2c.shape[1]
    return pl.pallas_call(
        functools.partial(_layer2_kernel, f=f),
        grid=(N // BLK,),
        in_specs=[
            pl.BlockSpec((N, din), lambda i: (0, 0)),
            pl.BlockSpec((din, f), lambda i: (0, 0)),
            pl.BlockSpec((2 * f, 1), lambda i: (0, 0)),
            pl.BlockSpec((BLK, N), lambda i: (i, 0)),
            pl.BlockSpec((f, out), lambda i: (0, 0)),
            pl.BlockSpec((1, out), lambda i: (0, 0)),
        ],
        out_specs=[
            pl.BlockSpec((BLK, N), lambda i: (i, 0)),
            pl.BlockSpec((BLK, out), lambda i: (i, 0)),
        ],
        out_shape=[
            jax.ShapeDtypeStruct((N, N), jnp.float32),
            jax.ShapeDtypeStruct((N, out), jnp.float32),
        ],
        scratch_shapes=_scratch(f),
    )(x, W, a, m8, e2c, b)


@jax.jit
def kernel(ft, adj, W1, a1, W2, a2, embd2class, bias):
    embd, m8 = _layer1(ft, W1, a1, adj)
    attention, logits = _layer2(embd, W2, a2, m8, embd2class, bias)
    return (logits, embd, attention)


# speedup vs baseline: 7.0996x; 1.0170x over previous
import functools

import jax
import jax.numpy as jnp
from jax.experimental import pallas as pl
from jax.experimental.pallas import tpu as pltpu

N = 4096
BLK = 512
ALPHA = 0.2
LOG2E = 1.4426950408889634


def _prep_to_scratch(x_ref, w_ref, a_ref, haug_s, c_s, d_s, f2_s, g_s,
                     cs_s, f):
    h = jnp.dot(x_ref[...], w_ref[...], preferred_element_type=jnp.float32)
    haug_s[:, :f] = h
    haug_s[:, f:] = jnp.ones((N, 1), jnp.float32)
    f1 = jnp.dot(h, a_ref[:f, :], preferred_element_type=jnp.float32)
    f2 = jax.lax.dot_general(
        a_ref[f:, :], h, (((0,), (1,)), ((), ())),
        preferred_element_type=jnp.float32)
    lg = jnp.float32(LOG2E)
    f2_s[...] = f2 * lg
    g_s[...] = f2 * jnp.float32(ALPHA * LOG2E)
    m2 = jnp.max(f2)
    t = f1 + m2
    m = jnp.maximum(t, t * jnp.float32(ALPHA))
    c_s[...] = (f1 - m) * lg
    d_s[...] = f1 * jnp.float32(ALPHA * LOG2E) - m * lg
    cs_s[...] = jnp.sum(h, axis=0, keepdims=True)


def _softmax_block(mask, i, haug_s, c_s, d_s, f2_s, g_s, cs_s, f):
    c = c_s[pl.ds(i * BLK, BLK), :]
    d = d_s[pl.ds(i * BLK, BLK), :]
    q = jnp.maximum(c + f2_s[...], d + g_s[...])
    pm = jnp.exp2(q) * mask
    hpz = jnp.dot(pm, haug_s[...], preferred_element_type=jnp.float32)
    s = hpz[:, f:]
    z = jnp.where(s == 0, jnp.float32(1.0), jnp.float32(0.0))
    r = jnp.float32(1.0) / (s + z * jnp.float32(N))
    hp = (hpz[:, :f] + z * cs_s[...]) * r
    return pm, z, r, hp


def _layer1_kernel(x_ref, w_ref, a_ref, adj_ref, out_ref, m8_ref,
                   haug_s, c_s, d_s, f2_s, g_s, cs_s, *, f):
    i = pl.program_id(0)

    @pl.when(i == 0)
    def _():
        _prep_to_scratch(x_ref, w_ref, a_ref, haug_s, c_s, d_s, f2_s,
                         g_s, cs_s, f)

    _, _, _, hp = _softmax_block(adj_ref[...], i, haug_s, c_s, d_s, f2_s,
                                 g_s, cs_s, f)
    out_ref[...] = jnp.maximum(hp, 0.0)
    m8_ref[...] = adj_ref[...].astype(jnp.int8)


def _layer2_kernel(x_ref, w_ref, a_ref, m8_ref, e2c_ref, b_ref,
                   attn_ref, logits_ref,
                   haug_s, c_s, d_s, f2_s, g_s, cs_s, *, f):
    i = pl.program_id(0)

    @pl.when(i == 0)
    def _():
        _prep_to_scratch(x_ref, w_ref, a_ref, haug_s, c_s, d_s, f2_s,
                         g_s, cs_s, f)

    pm, z, r, hp = _softmax_block(m8_ref[...].astype(jnp.float32), i,
                                  haug_s, c_s, d_s, f2_s, g_s, cs_s, f)
    attn_ref[...] = pm * r

    @pl.when(jnp.max(z) > 0)
    def _():
        attn_ref[...] = attn_ref[...] + z * r
    logits_ref[...] = (
        jnp.dot(hp, e2c_ref[...], preferred_element_type=jnp.float32)
        + b_ref[...])


def _scratch(f):
    return [
        pltpu.VMEM((N, f + 1), jnp.float32),
        pltpu.VMEM((N, 1), jnp.float32),
        pltpu.VMEM((N, 1), jnp.float32),
        pltpu.VMEM((1, N), jnp.float32),
        pltpu.VMEM((1, N), jnp.float32),
        pltpu.VMEM((1, f), jnp.float32),
    ]


def _layer1(ft, W, a, adj):
    f = W.shape[1]
    din = ft.shape[1]
    return pl.pallas_call(
        functools.partial(_layer1_kernel, f=f),
        grid=(N // BLK,),
        in_specs=[
            pl.BlockSpec((N, din), lambda i: (0, 0)),
            pl.BlockSpec((din, f), lambda i: (0, 0)),
            pl.BlockSpec((2 * f, 1), lambda i: (0, 0)),
            pl.BlockSpec((BLK, N), lambda i: (i, 0)),
        ],
        out_specs=[
            pl.BlockSpec((BLK, f), lambda i: (i, 0)),
            pl.BlockSpec((BLK, N), lambda i: (i, 0)),
        ],
        out_shape=[
            jax.ShapeDtypeStruct((N, f), jnp.float32),
            jax.ShapeDtypeStruct((N, N), jnp.int8),
        ],
        scratch_shapes=_scratch(f),
    )(ft, W, a, adj)


def _layer2(x, W, a, m8, e2c, b):
    f = W.shape[1]
    din = x.shape[1]
    out = e2c.shape[1]
    return pl.pallas_call(
        functools.partial(_layer2_kernel, f=f),
        grid=(N // BLK,),
        in_specs=[
            pl.BlockSpec((N, din), lambda i: (0, 0)),
            pl.BlockSpec((din, f), lambda i: (0, 0)),
            pl.BlockSpec((2 * f, 1), lambda i: (0, 0)),
            pl.BlockSpec((BLK, N), lambda i: (i, 0)),
            pl.BlockSpec((f, out), lambda i: (0, 0)),
            pl.BlockSpec((1, out), lambda i: (0, 0)),
        ],
        out_specs=[
            pl.BlockSpec((BLK, N), lambda i: (i, 0)),
            pl.BlockSpec((BLK, out), lambda i: (i, 0)),
        ],
        out_shape=[
            jax.ShapeDtypeStruct((N, N), jnp.float32),
            jax.ShapeDtypeStruct((N, out), jnp.float32),
        ],
        scratch_shapes=_scratch(f),
    )(x, W, a, m8, e2c, b)


@jax.jit
def kernel(ft, adj, W1, a1, W2, a2, embd2class, bias):
    embd, m8 = _layer1(ft, W1, a1, adj)
    attention, logits = _layer2(embd, W2, a2, m8, embd2class, bias)
    return (logits, embd, attention)


# speedup vs baseline: 7.2236x; 1.0175x over previous
import functools

import jax
import jax.numpy as jnp
from jax.experimental import pallas as pl
from jax.experimental.pallas import tpu as pltpu

N = 4096
BLK = 256
NB = N // BLK
ALPHA = 0.2
LOG2E = 1.4426950408889634


def _prep_to_scratch(x, w_ref, a_ref, haug_s, c_s, d_s, f2_s, g_s, cs_s, f):
    h = jnp.dot(x, w_ref[...], preferred_element_type=jnp.float32)
    haug_s[:, :f] = h
    haug_s[:, f:] = jnp.ones((N, 1), jnp.float32)
    f1 = jnp.dot(h, a_ref[:f, :], preferred_element_type=jnp.float32)
    f2 = jax.lax.dot_general(
        a_ref[f:, :], h, (((0,), (1,)), ((), ())),
        preferred_element_type=jnp.float32)
    lg = jnp.float32(LOG2E)
    f2_s[...] = f2 * lg
    g_s[...] = f2 * jnp.float32(ALPHA * LOG2E)
    m2 = jnp.max(f2)
    t = f1 + m2
    m = jnp.maximum(t, t * jnp.float32(ALPHA))
    c_s[...] = (f1 - m) * lg
    d_s[...] = f1 * jnp.float32(ALPHA * LOG2E) - m * lg
    cs_s[...] = jnp.sum(h, axis=0, keepdims=True)


def _softmax_block(mask, j, haug_s, c_s, d_s, f2_s, g_s, cs_s, f):
    c = c_s[pl.ds(j * BLK, BLK), :]
    d = d_s[pl.ds(j * BLK, BLK), :]
    q = jnp.maximum(c + f2_s[...], d + g_s[...])
    pm = jnp.exp2(q) * mask
    hpz = jnp.dot(pm, haug_s[...], preferred_element_type=jnp.float32)
    s = hpz[:, f:]
    z = jnp.where(s == 0, jnp.float32(1.0), jnp.float32(0.0))
    r = jnp.float32(1.0) / (s + z * jnp.float32(N))
    hp = (hpz[:, :f] + z * cs_s[...]) * r
    return pm, z, r, hp


def _gat_kernel(ft_ref, w1_ref, a1_ref, w2_ref, a2_ref, e2c_ref, b_ref,
                adj_ref, embd_ref, attn_ref, logits_ref,
                haug_s, c_s, d_s, f2_s, g_s, cs_s, m8_s, embd_s, *, f):
    i = pl.program_id(0)

    @pl.when(i == 0)
    def _():
        _prep_to_scratch(ft_ref[...], w1_ref, a1_ref, haug_s, c_s, d_s,
                         f2_s, g_s, cs_s, f)

    @pl.when(i < NB)
    def _():
        j = i
        adj = adj_ref[...]
        m8_s[pl.ds(j * BLK, BLK), :] = adj.astype(jnp.int8)
        _, _, _, hp = _softmax_block(adj, j, haug_s, c_s, d_s, f2_s, g_s,
                                     cs_s, f)
        embd = jnp.maximum(hp, 0.0)
        embd_ref[...] = embd
        embd_s[pl.ds(j * BLK, BLK), :] = embd

    @pl.when(i == NB)
    def _():
        _prep_to_scratch(embd_s[...], w2_ref, a2_ref, haug_s, c_s, d_s,
                         f2_s, g_s, cs_s, f)

    @pl.when(i >= NB)
    def _():
        j = i - NB
        mask = m8_s[pl.ds(j * BLK, BLK), :].astype(jnp.float32)
        pm, z, r, hp = _softmax_block(mask, j, haug_s, c_s, d_s, f2_s,
                                      g_s, cs_s, f)
        attn_ref[...] = pm * r

        @pl.when(jnp.max(z) > 0)
        def _():
            attn_ref[...] = attn_ref[...] + z * r

        logits_ref[...] = (
            jnp.dot(hp, e2c_ref[...], preferred_element_type=jnp.float32)
            + b_ref[...])


def _gat(ft, adj, W1, a1, W2, a2, e2c, b):
    f = W1.shape[1]
    din = ft.shape[1]
    out = e2c.shape[1]
    last = NB - 1
    return pl.pallas_call(
        functools.partial(_gat_kernel, f=f),
        grid=(2 * NB,),
        in_specs=[
            pl.BlockSpec((N, din), lambda i: (0, 0)),
            pl.BlockSpec((din, f), lambda i: (0, 0)),
            pl.BlockSpec((2 * f, 1), lambda i: (0, 0)),
            pl.BlockSpec((f, f), lambda i: (0, 0)),
            pl.BlockSpec((2 * f, 1), lambda i: (0, 0)),
            pl.BlockSpec((f, out), lambda i: (0, 0)),
            pl.BlockSpec((1, out), lambda i: (0, 0)),
            pl.BlockSpec((BLK, N), lambda i: (jnp.minimum(i, last), 0)),
        ],
        out_specs=[
            pl.BlockSpec((BLK, f), lambda i: (jnp.minimum(i, last), 0)),
            pl.BlockSpec((BLK, N), lambda i: (jnp.maximum(i - NB, 0), 0)),
            pl.BlockSpec((BLK, out), lambda i: (jnp.maximum(i - NB, 0), 0)),
        ],
        out_shape=[
            jax.ShapeDtypeStruct((N, f), jnp.float32),
            jax.ShapeDtypeStruct((N, N), jnp.float32),
            jax.ShapeDtypeStruct((N, out), jnp.float32),
        ],
        scratch_shapes=[
            pltpu.VMEM((N, f + 1), jnp.float32),
            pltpu.VMEM((N, 1), jnp.float32),
            pltpu.VMEM((N, 1), jnp.float32),
            pltpu.VMEM((1, N), jnp.float32),
            pltpu.VMEM((1, N), jnp.float32),
            pltpu.VMEM((1, f), jnp.float32),
            pltpu.VMEM((N, N), jnp.int8),
            pltpu.VMEM((N, f), jnp.float32),
        ],
    )(ft, W1, a1, W2, a2, e2c, b, adj)


@jax.jit
def kernel(ft, adj, W1, a1, W2, a2, embd2class, bias):
    embd, attention, logits = _gat(ft, adj, W1, a1, W2, a2,
                                   embd2class, bias)
    return (logits, embd, attention)


# speedup vs baseline: 7.7254x; 1.0695x over previous
import functools

import jax
import jax.numpy as jnp
from jax.experimental import pallas as pl
from jax.experimental.pallas import tpu as pltpu

N = 4096
BLK1 = 512
BLK2 = 256
NB1 = N // BLK1
NB2 = N // BLK2
ALPHA = 0.2
LOG2E = 1.4426950408889634


def _prep_to_scratch(x, w_ref, a_ref, haug_s, c_s, d_s, f2_s, g_s, cs_s, f):
    h = jnp.dot(x, w_ref[...], preferred_element_type=jnp.float32)
    haug_s[:, :f] = h
    haug_s[:, f:] = jnp.ones((N, 1), jnp.float32)
    f1 = jnp.dot(h, a_ref[:f, :], preferred_element_type=jnp.float32)
    f2 = jax.lax.dot_general(
        a_ref[f:, :], h, (((0,), (1,)), ((), ())),
        preferred_element_type=jnp.float32)
    lg = jnp.float32(LOG2E)
    f2_s[...] = f2 * lg
    g_s[...] = f2 * jnp.float32(ALPHA * LOG2E)
    m2 = jnp.max(f2)
    t = f1 + m2
    m = jnp.maximum(t, t * jnp.float32(ALPHA))
    c_s[...] = (f1 - m) * lg
    d_s[...] = f1 * jnp.float32(ALPHA * LOG2E) - m * lg
    cs_s[...] = jnp.sum(h, axis=0, keepdims=True)


def _softmax_block(mask, j, blk, haug_s, c_s, d_s, f2_s, g_s, cs_s, f):
    c = c_s[pl.ds(j * blk, blk), :]
    d = d_s[pl.ds(j * blk, blk), :]
    q = jnp.maximum(c + f2_s[...], d + g_s[...])
    pm = jnp.exp2(q) * mask
    hpz = jnp.dot(pm, haug_s[...], preferred_element_type=jnp.float32)
    s = hpz[:, f:]
    z = jnp.where(s == 0, jnp.float32(1.0), jnp.float32(0.0))
    r = jnp.float32(1.0) / (s + z * jnp.float32(N))
    hp = (hpz[:, :f] + z * cs_s[...]) * r
    return pm, z, r, hp


def _gat_kernel(ft_ref, w1_ref, a1_ref, w2_ref, a2_ref, e2c_ref, b_ref,
                adj_ref, embd_ref, attn_ref, logits_ref,
                haug_s, c_s, d_s, f2_s, g_s, cs_s, m8_s, embd_s, *, f):
    i = pl.program_id(0)

    @pl.when(i == 0)
    def _():
        _prep_to_scratch(ft_ref[...], w1_ref, a1_ref, haug_s, c_s, d_s,
                         f2_s, g_s, cs_s, f)

    @pl.when(i < NB1)
    def _():
        j = i
        adj = adj_ref[...]
        m8_s[pl.ds(j * BLK1, BLK1), :] = adj.astype(jnp.int8)
        _, _, _, hp = _softmax_block(adj, j, BLK1, haug_s, c_s, d_s, f2_s,
                                     g_s, cs_s, f)
        embd = jnp.maximum(hp, 0.0)
        embd_ref[...] = embd
        embd_s[pl.ds(j * BLK1, BLK1), :] = embd

    @pl.when(i == NB1)
    def _():
        _prep_to_scratch(embd_s[...], w2_ref, a2_ref, haug_s, c_s, d_s,
                         f2_s, g_s, cs_s, f)

    @pl.when(i >= NB1)
    def _():
        j = i - NB1
        mask = m8_s[pl.ds(j * BLK2, BLK2), :].astype(jnp.float32)
        pm, z, r, hp = _softmax_block(mask, j, BLK2, haug_s, c_s, d_s,
                                      f2_s, g_s, cs_s, f)
        attn_ref[...] = pm * r

        @pl.when(jnp.max(z) > 0)
        def _():
            attn_ref[...] = attn_ref[...] + z * r

        logits_ref[...] = (
            jnp.dot(hp, e2c_ref[...], preferred_element_type=jnp.float32)
            + b_ref[...])


def _gat(ft, adj, W1, a1, W2, a2, e2c, b):
    f = W1.shape[1]
    din = ft.shape[1]
    out = e2c.shape[1]
    last = NB1 - 1
    return pl.pallas_call(
        functools.partial(_gat_kernel, f=f),
        grid=(NB1 + NB2,),
        in_specs=[
            pl.BlockSpec((N, din), lambda i: (0, 0)),
            pl.BlockSpec((din, f), lambda i: (0, 0)),
            pl.BlockSpec((2 * f, 1), lambda i: (0, 0)),
            pl.BlockSpec((f, f), lambda i: (0, 0)),
            pl.BlockSpec((2 * f, 1), lambda i: (0, 0)),
            pl.BlockSpec((f, out), lambda i: (0, 0)),
            pl.BlockSpec((1, out), lambda i: (0, 0)),
            pl.BlockSpec((BLK1, N), lambda i: (jnp.minimum(i, last), 0)),
        ],
        out_specs=[
            pl.BlockSpec((BLK1, f), lambda i: (jnp.minimum(i, last), 0)),
            pl.BlockSpec((BLK2, N), lambda i: (jnp.maximum(i - NB1, 0), 0)),
            pl.BlockSpec((BLK2, out),
                         lambda i: (jnp.maximum(i - NB1, 0), 0)),
        ],
        out_shape=[
            jax.ShapeDtypeStruct((N, f), jnp.float32),
            jax.ShapeDtypeStruct((N, N), jnp.float32),
            jax.ShapeDtypeStruct((N, out), jnp.float32),
        ],
        scratch_shapes=[
            pltpu.VMEM((N, f + 1), jnp.float32),
            pltpu.VMEM((N, 1), jnp.float32),
            pltpu.VMEM((N, 1), jnp.float32),
            pltpu.VMEM((1, N), jnp.float32),
            pltpu.VMEM((1, N), jnp.float32),
            pltpu.VMEM((1, f), jnp.float32),
            pltpu.VMEM((N, N), jnp.int8),
            pltpu.VMEM((N, f), jnp.float32),
        ],
    )(ft, W1, a1, W2, a2, e2c, b, adj)


@jax.jit
def kernel(ft, adj, W1, a1, W2, a2, embd2class, bias):
    embd, attention, logits = _gat(ft, adj, W1, a1, W2, a2,
                                   embd2class, bias)
    return (logits, embd, attention)
